# Initial kernel scaffold; baseline (speedup 1.0000x reference)
import jax
import jax.numpy as jnp
from jax.experimental import pallas as pl


def kernel(x, edge_index, batch, enc_W1, enc_b1, enc_W2, enc_b2, enc_W3, enc_b3, dec_Wt, dec_bt, dec_W1, dec_b1, dec_W2, dec_b2, dec_W3, dec_b3, dec_Wout, dec_bout):
    raise NotImplementedError("write your pallas kernel here")


# speedup vs baseline: 12.5986x; 12.5986x over previous
import functools
import numpy as np
import jax
import jax.numpy as jnp
from jax import lax
from jax.experimental import pallas as pl
from jax.experimental.pallas import tpu as pltpu
from jax.experimental.pallas import tpu_sc as plsc

N = 10000
E = 320000
D = 128
H = 64
B = 64
NDEC = 100
NPAD = 112
OUT = 128

NC = 2
NS = 16
NW = NC * NS
EPW = E // NW
CH = 80
NIT = EPW // CH
NA = 10240
RPT = NA // NS


def _dec_edges_np(b, n):
    i = np.arange(n)
    p1 = np.stack([i[:-1], i[:-1] + 1], 1)
    p2 = np.stack([i[:-2], i[:-2] + 2], 1)
    pairs = np.concatenate([p1, p2], 0)
    both = np.concatenate([pairs, pairs[:, ::-1]], 0)
    offs = (np.arange(b) * n)[:, None, None]
    return (both[None] + offs).reshape(-1, 2).T


def _dec_ahat_np(n, npad):
    i = np.arange(n)
    p1 = np.stack([i[:-1], i[:-1] + 1], 1)
    p2 = np.stack([i[:-2], i[:-2] + 2], 1)
    pairs = np.concatenate([p1, p2], 0)
    both = np.concatenate([pairs, pairs[:, ::-1]], 0)
    A = np.zeros((n, n), np.float32)
    A[both[:, 1], both[:, 0]] += 1.0
    A += np.eye(n, dtype=np.float32)
    dinv = 1.0 / np.sqrt(A.sum(1))
    Ah = dinv[:, None] * A * dinv[None, :]
    Ap = np.zeros((npad, npad), np.float32)
    Ap[:n, :n] = Ah
    return Ap


_DEC_EI_NP = _dec_edges_np(B, NDEC).astype(np.int32)
_AHAT_NP = _dec_ahat_np(NDEC, NPAD)

_sc_mesh = plsc.VectorSubcoreMesh(core_axis_name="c", subcore_axis_name="s")


@functools.partial(
    pl.kernel,
    mesh=_sc_mesh,
    out_type=jax.ShapeDtypeStruct((NC, NA, 8), jnp.float32),
    compiler_params=pltpu.CompilerParams(use_tc_tiling_on_sc=False),
    scratch_types=[
        pltpu.VMEM((CH,), jnp.int32),
        pltpu.VMEM((CH, 8), jnp.float32),
        pltpu.VMEM_SHARED((NA, 8), jnp.float32),
    ],
)
def _deg_sc(dst_hbm, ones_hbm, zeros_hbm, out_hbm, dst_v, ones_v, acc):
    cid = lax.axis_index("c")
    sid = lax.axis_index("s")
    wid = sid * NC + cid
    r0 = sid * RPT
    pltpu.sync_copy(zeros_hbm.at[pl.ds(r0, RPT)], acc.at[pl.ds(r0, RPT)])
    pltpu.sync_copy(ones_hbm, ones_v)
    plsc.subcore_barrier()
    base = wid * EPW

    def body(it, _):
        off = pl.multiple_of(base + it * CH, 8)
        pltpu.sync_copy(dst_hbm.at[pl.ds(off, CH)], dst_v)
        pltpu.sync_copy(ones_v, acc.at[dst_v], add=True)
        return ()

    lax.fori_loop(0, NIT, body, ())
    plsc.subcore_barrier()
    pltpu.sync_copy(acc.at[pl.ds(r0, RPT)], out_hbm.at[cid, pl.ds(r0, RPT)])


@functools.partial(
    pl.kernel,
    mesh=_sc_mesh,
    out_type=jax.ShapeDtypeStruct((NC, NA, H), jnp.float32),
    compiler_params=pltpu.CompilerParams(use_tc_tiling_on_sc=False),
    scratch_types=[
        pltpu.VMEM((CH,), jnp.int32),
        pltpu.VMEM((CH,), jnp.int32),
        pltpu.VMEM((CH, H), jnp.float32),
        pltpu.VMEM_SHARED((NA, H), jnp.float32),
        pltpu.SemaphoreType.DMA,
    ],
)
def _scat_sc(g_hbm, src_hbm, dst_hbm, zeros_hbm, out_hbm,
             src_v, dst_v, rows_v, acc, sem):
    cid = lax.axis_index("c")
    sid = lax.axis_index("s")
    wid = sid * NC + cid
    r0 = sid * RPT
    pltpu.sync_copy(zeros_hbm.at[pl.ds(r0, RPT)], acc.at[pl.ds(r0, RPT)])
    plsc.subcore_barrier()
    base = wid * EPW

    def body(it, _):
        off = pl.multiple_of(base + it * CH, 8)
        pltpu.sync_copy(src_hbm.at[pl.ds(off, CH)], src_v)
        pltpu.async_copy(g_hbm.at[src_v], rows_v, sem).wait()
        pltpu.sync_copy(dst_hbm.at[pl.ds(off, CH)], dst_v)
        pltpu.sync_copy(rows_v, acc.at[dst_v], add=True)
        return ()

    lax.fori_loop(0, NIT, body, ())
    plsc.subcore_barrier()
    pltpu.sync_copy(acc.at[pl.ds(r0, RPT)], out_hbm.at[cid, pl.ds(r0, RPT)])


def _tca_body(deg_ref, x_ref, w_ref, dinv_ref, g_ref):
    deg = 1.0 + deg_ref[0][:N, 0:1] + deg_ref[1][:N, 0:1]
    dinv = lax.rsqrt(deg)
    dinv_ref[...] = jnp.broadcast_to(dinv, (N, 8))
    g_ref[...] = dinv * jnp.dot(x_ref[...], w_ref[...],
                                preferred_element_type=jnp.float32)


_tca = pl.pallas_call(
    _tca_body,
    out_shape=(jax.ShapeDtypeStruct((N, 8), jnp.float32),
               jax.ShapeDtypeStruct((N, H), jnp.float32)),
)


def _tcb_body(s_ref, g_ref, dinv_ref, b_ref, w_ref, out_ref):
    dinv = dinv_ref[:, 0:1]
    t = dinv * (s_ref[0][:N] + s_ref[1][:N] + g_ref[...]) + b_ref[...]
    h = jnp.maximum(t, 0.0)
    out_ref[...] = dinv * jnp.dot(h, w_ref[...],
                                  preferred_element_type=jnp.float32)


_tcb = pl.pallas_call(
    _tcb_body,
    out_shape=jax.ShapeDtypeStruct((N, H), jnp.float32),
)


def _tcd_body(s_ref, g_ref, dinv_ref, b_ref, batch_ref, emb_ref):
    dinv = dinv_ref[:, 0:1]
    hf = dinv * (s_ref[0][:N] + s_ref[1][:N] + g_ref[...]) + b_ref[...]
    seg = lax.broadcasted_iota(jnp.int32, (B, N), 0)
    onehot = (batch_ref[...] == seg).astype(jnp.float32)
    ssum = jnp.dot(onehot, hf, preferred_element_type=jnp.float32)
    cnt = jnp.sum(onehot, axis=1, keepdims=True)
    emb_ref[...] = ssum / jnp.maximum(cnt, 1.0)


_tcd = pl.pallas_call(
    _tcd_body,
    out_shape=jax.ShapeDtypeStruct((B, H), jnp.float32),
)


def _dec_body(emb_ref, a_ref, wt_ref, bt_ref, w1_ref, b1_ref, w2_ref,
              b2_ref, w3_ref, b3_ref, wo_ref, bo_ref, out_ref):
    init = jnp.dot(emb_ref[0], wt_ref[...],
                   preferred_element_type=jnp.float32) + bt_ref[...]
    z = jnp.broadcast_to(init, (NPAD, H))
    A = a_ref[...]
    for w_ref, b_ref in ((w1_ref, b1_ref), (w2_ref, b2_ref), (w3_ref, b3_ref)):
        t = jnp.dot(z, w_ref[...], preferred_element_type=jnp.float32)
        z = jnp.maximum(jnp.dot(A, t, preferred_element_type=jnp.float32)
                        + b_ref[...], 0.0)
    out_ref[0] = jnp.dot(z, wo_ref[...],
                         preferred_element_type=jnp.float32) + bo_ref[...]


def _full(shape):
    nd = len(shape)
    return pl.BlockSpec(shape, lambda b, _nd=nd: (0,) * _nd)


_dec = pl.pallas_call(
    _dec_body,
    grid=(B,),
    in_specs=[
        pl.BlockSpec((1, 1, H), lambda b: (b, 0, 0)),
        _full((NPAD, NPAD)),
        _full((H, H)), _full((1, H)),
        _full((H, H)), _full((1, H)),
        _full((H, H)), _full((1, H)),
        _full((H, H)), _full((1, H)),
        _full((H, OUT)), _full((1, OUT)),
    ],
    out_specs=pl.BlockSpec((1, NPAD, OUT), lambda b: (b, 0, 0)),
    out_shape=jax.ShapeDtypeStruct((B, NPAD, OUT), jnp.float32),
)


@jax.jit
def kernel(x, edge_index, batch, enc_W1, enc_b1, enc_W2, enc_b2, enc_W3,
           enc_b3, dec_Wt, dec_bt, dec_W1, dec_b1, dec_W2, dec_b2, dec_W3,
           dec_b3, dec_Wout, dec_bout):
    src = edge_index[0]
    dst = edge_index[1]
    zeros8 = jnp.zeros((NA, 8), jnp.float32)
    ones8 = jnp.ones((CH, 8), jnp.float32)
    zeros64 = jnp.zeros((NA, H), jnp.float32)

    deg2 = _deg_sc(dst, ones8, zeros8)
    dinv8, g1 = _tca(deg2, x, enc_W1)
    s1 = _scat_sc(g1, src, dst, zeros64)
    g2 = _tcb(s1, g1, dinv8, enc_b1.reshape(1, H), enc_W2)
    s2 = _scat_sc(g2, src, dst, zeros64)
    g3 = _tcb(s2, g2, dinv8, enc_b2.reshape(1, H), enc_W3)
    s3 = _scat_sc(g3, src, dst, zeros64)
    emb = _tcd(s3, g3, dinv8, enc_b3.reshape(1, H), batch.reshape(1, N))

    node_pad = _dec(emb.reshape(B, 1, H), jnp.asarray(_AHAT_NP),
                    dec_Wt, dec_bt.reshape(1, H),
                    dec_W1, dec_b1.reshape(1, H),
                    dec_W2, dec_b2.reshape(1, H),
                    dec_W3, dec_b3.reshape(1, H),
                    dec_Wout, dec_bout.reshape(1, OUT))
    node_out = node_pad[:, :NDEC, :]
    return (node_out, emb, jnp.asarray(_DEC_EI_NP))


# speedup vs baseline: 30.6266x; 2.4310x over previous
import functools
import numpy as np
import jax
import jax.numpy as jnp
from jax import lax
from jax.experimental import pallas as pl
from jax.experimental.pallas import tpu as pltpu
from jax.experimental.pallas import tpu_sc as plsc

N = 10000
E = 320000
D = 128
H = 64
B = 64
NDEC = 100
NPAD = 112
OUT = 128

NC = 2
NS = 16
NW = NC * NS
EPW = E // NW
CH = 125
NIT = EPW // CH
NPAIR = NIT // 2
NA = 10240
RPT = NA // NS


def _dec_edges_np(b, n):
    i = np.arange(n)
    p1 = np.stack([i[:-1], i[:-1] + 1], 1)
    p2 = np.stack([i[:-2], i[:-2] + 2], 1)
    pairs = np.concatenate([p1, p2], 0)
    both = np.concatenate([pairs, pairs[:, ::-1]], 0)
    offs = (np.arange(b) * n)[:, None, None]
    return (both[None] + offs).reshape(-1, 2).T


def _dec_ahat_np(n, npad):
    i = np.arange(n)
    p1 = np.stack([i[:-1], i[:-1] + 1], 1)
    p2 = np.stack([i[:-2], i[:-2] + 2], 1)
    pairs = np.concatenate([p1, p2], 0)
    both = np.concatenate([pairs, pairs[:, ::-1]], 0)
    A = np.zeros((n, n), np.float32)
    A[both[:, 1], both[:, 0]] += 1.0
    A += np.eye(n, dtype=np.float32)
    dinv = 1.0 / np.sqrt(A.sum(1))
    Ah = dinv[:, None] * A * dinv[None, :]
    Ap = np.zeros((npad, npad), np.float32)
    Ap[:n, :n] = Ah
    return Ap


_DEC_EI_NP = _dec_edges_np(B, NDEC).astype(np.int32)
_AHAT_NP = _dec_ahat_np(NDEC, NPAD)

_sc_mesh = plsc.VectorSubcoreMesh(core_axis_name="c", subcore_axis_name="s")


@functools.partial(
    pl.kernel,
    mesh=_sc_mesh,
    out_type=jax.ShapeDtypeStruct((NC, NA, 8), jnp.float32),
    compiler_params=pltpu.CompilerParams(use_tc_tiling_on_sc=False),
    scratch_types=[
        pltpu.VMEM((NIT, CH), jnp.int32),
        pltpu.VMEM((CH, 8), jnp.float32),
        pltpu.VMEM_SHARED((NA, 8), jnp.float32),
    ],
)
def _deg_sc(dst_hbm, ones_hbm, zeros_hbm, out_hbm, dst_v, ones_v, acc):
    cid = lax.axis_index("c")
    sid = lax.axis_index("s")
    wid = sid * NC + cid
    r0 = sid * RPT
    pltpu.sync_copy(zeros_hbm.at[pl.ds(r0, RPT)], acc.at[pl.ds(r0, RPT)])
    pltpu.sync_copy(ones_hbm, ones_v)
    pltpu.sync_copy(dst_hbm.at[wid], dst_v)
    plsc.subcore_barrier()

    def body(it, _):
        pltpu.sync_copy(ones_v, acc.at[dst_v.at[it]], add=True)
        return ()

    lax.fori_loop(0, NIT, body, ())
    plsc.subcore_barrier()
    pltpu.sync_copy(acc.at[pl.ds(r0, RPT)], out_hbm.at[cid, pl.ds(r0, RPT)])


@functools.partial(
    pl.kernel,
    mesh=_sc_mesh,
    out_type=jax.ShapeDtypeStruct((NC, NA, H), jnp.float32),
    compiler_params=pltpu.CompilerParams(use_tc_tiling_on_sc=False),
    scratch_types=[
        pltpu.VMEM((NIT, CH), jnp.int32),
        pltpu.VMEM((NIT, CH), jnp.int32),
        pltpu.VMEM((CH, H), jnp.float32),
        pltpu.VMEM((CH, H), jnp.float32),
        pltpu.VMEM_SHARED((NA, H), jnp.float32),
        pltpu.SemaphoreType.DMA,
        pltpu.SemaphoreType.DMA,
    ],
)
def _scat_sc(g_hbm, src_hbm, dst_hbm, zeros_hbm, out_hbm,
             src_v, dst_v, rows0, rows1, acc, sem0, sem1):
    cid = lax.axis_index("c")
    sid = lax.axis_index("s")
    wid = sid * NC + cid
    r0 = sid * RPT
    pltpu.sync_copy(zeros_hbm.at[pl.ds(r0, RPT)], acc.at[pl.ds(r0, RPT)])
    pltpu.sync_copy(src_hbm.at[wid], src_v)
    pltpu.sync_copy(dst_hbm.at[wid], dst_v)
    plsc.subcore_barrier()
    pltpu.async_copy(g_hbm.at[src_v.at[0]], rows0, sem0)

    def body(j, _):
        i0 = 2 * j
        pltpu.async_copy(g_hbm.at[src_v.at[i0 + 1]], rows1, sem1)
        pltpu.make_async_copy(g_hbm.at[src_v.at[i0]], rows0, sem0).wait()
        pltpu.sync_copy(rows0, acc.at[dst_v.at[i0]], add=True)

        @pl.when(j < NPAIR - 1)
        def _():
            pltpu.async_copy(g_hbm.at[src_v.at[i0 + 2]], rows0, sem0)

        pltpu.make_async_copy(g_hbm.at[src_v.at[i0 + 1]], rows1, sem1).wait()
        pltpu.sync_copy(rows1, acc.at[dst_v.at[i0 + 1]], add=True)
        return ()

    lax.fori_loop(0, NPAIR, body, ())
    plsc.subcore_barrier()
    pltpu.sync_copy(acc.at[pl.ds(r0, RPT)], out_hbm.at[cid, pl.ds(r0, RPT)])


def _tca_body(deg_ref, x_ref, w_ref, dinv_ref, g_ref):
    deg = 1.0 + deg_ref[0][:N, 0:1] + deg_ref[1][:N, 0:1]
    dinv = lax.rsqrt(deg)
    dinv_ref[...] = jnp.broadcast_to(dinv, (N, 8))
    g_ref[...] = dinv * jnp.dot(x_ref[...], w_ref[...],
                                preferred_element_type=jnp.float32)


_tca = pl.pallas_call(
    _tca_body,
    out_shape=(jax.ShapeDtypeStruct((N, 8), jnp.float32),
               jax.ShapeDtypeStruct((N, H), jnp.float32)),
)


def _tcb_body(s_ref, g_ref, dinv_ref, b_ref, w_ref, out_ref):
    dinv = dinv_ref[:, 0:1]
    t = dinv * (s_ref[0][:N] + s_ref[1][:N] + g_ref[...]) + b_ref[...]
    h = jnp.maximum(t, 0.0)
    out_ref[...] = dinv * jnp.dot(h, w_ref[...],
                                  preferred_element_type=jnp.float32)


_tcb = pl.pallas_call(
    _tcb_body,
    out_shape=jax.ShapeDtypeStruct((N, H), jnp.float32),
)


def _tcd_body(s_ref, g_ref, dinv_ref, b_ref, batch_ref, emb_ref):
    dinv = dinv_ref[:, 0:1]
    hf = dinv * (s_ref[0][:N] + s_ref[1][:N] + g_ref[...]) + b_ref[...]
    seg = lax.broadcasted_iota(jnp.int32, (B, N), 0)
    onehot = (batch_ref[...] == seg).astype(jnp.float32)
    ssum = jnp.dot(onehot, hf, preferred_element_type=jnp.float32)
    cnt = jnp.sum(onehot, axis=1, keepdims=True)
    emb_ref[...] = ssum / jnp.maximum(cnt, 1.0)


_tcd = pl.pallas_call(
    _tcd_body,
    out_shape=jax.ShapeDtypeStruct((B, H), jnp.float32),
)


def _dec_body(emb_ref, a_ref, wt_ref, bt_ref, w1_ref, b1_ref, w2_ref,
              b2_ref, w3_ref, b3_ref, wo_ref, bo_ref, out_ref):
    init = jnp.dot(emb_ref[0], wt_ref[...],
                   preferred_element_type=jnp.float32) + bt_ref[...]
    z = jnp.broadcast_to(init, (NPAD, H))
    A = a_ref[...]
    for w_ref, b_ref in ((w1_ref, b1_ref), (w2_ref, b2_ref), (w3_ref, b3_ref)):
        t = jnp.dot(z, w_ref[...], preferred_element_type=jnp.float32)
        z = jnp.maximum(jnp.dot(A, t, preferred_element_type=jnp.float32)
                        + b_ref[...], 0.0)
    out_ref[0] = jnp.dot(z, wo_ref[...],
                         preferred_element_type=jnp.float32) + bo_ref[...]


def _full(shape):
    nd = len(shape)
    return pl.BlockSpec(shape, lambda b, _nd=nd: (0,) * _nd)


_dec = pl.pallas_call(
    _dec_body,
    grid=(B,),
    in_specs=[
        pl.BlockSpec((1, 1, H), lambda b: (b, 0, 0)),
        _full((NPAD, NPAD)),
        _full((H, H)), _full((1, H)),
        _full((H, H)), _full((1, H)),
        _full((H, H)), _full((1, H)),
        _full((H, H)), _full((1, H)),
        _full((H, OUT)), _full((1, OUT)),
    ],
    out_specs=pl.BlockSpec((1, NPAD, OUT), lambda b: (b, 0, 0)),
    out_shape=jax.ShapeDtypeStruct((B, NPAD, OUT), jnp.float32),
)


@jax.jit
def kernel(x, edge_index, batch, enc_W1, enc_b1, enc_W2, enc_b2, enc_W3,
           enc_b3, dec_Wt, dec_bt, dec_W1, dec_b1, dec_W2, dec_b2, dec_W3,
           dec_b3, dec_Wout, dec_bout):
    src = edge_index[0].reshape(NW, NIT, CH)
    dst = edge_index[1].reshape(NW, NIT, CH)
    zeros8 = jnp.zeros((NA, 8), jnp.float32)
    ones8 = jnp.ones((CH, 8), jnp.float32)
    zeros64 = jnp.zeros((NA, H), jnp.float32)

    deg2 = _deg_sc(dst, ones8, zeros8)
    dinv8, g1 = _tca(deg2, x, enc_W1)
    s1 = _scat_sc(g1, src, dst, zeros64)
    g2 = _tcb(s1, g1, dinv8, enc_b1.reshape(1, H), enc_W2)
    s2 = _scat_sc(g2, src, dst, zeros64)
    g3 = _tcb(s2, g2, dinv8, enc_b2.reshape(1, H), enc_W3)
    s3 = _scat_sc(g3, src, dst, zeros64)
    emb = _tcd(s3, g3, dinv8, enc_b3.reshape(1, H), batch.reshape(1, N))

    node_pad = _dec(emb.reshape(B, 1, H), jnp.asarray(_AHAT_NP),
                    dec_Wt, dec_bt.reshape(1, H),
                    dec_W1, dec_b1.reshape(1, H),
                    dec_W2, dec_b2.reshape(1, H),
                    dec_W3, dec_b3.reshape(1, H),
                    dec_Wout, dec_bout.reshape(1, OUT))
    node_out = node_pad[:, :NDEC, :]
    return (node_out, emb, jnp.asarray(_DEC_EI_NP))
